# Initial kernel scaffold; baseline (speedup 1.0000x reference)
import jax
import jax.numpy as jnp
from jax.experimental import pallas as pl


def kernel(x, edge_index, Wq1, bq1, Wk1, bk1, Wv1, bv1, Ws1, bs1, Wq2, bq2, Wk2, bk2, Wv2, bv2, Ws2, bs2):
    raise NotImplementedError("write your pallas kernel here")


# speedup vs baseline: 12.9902x; 12.9902x over previous
import functools

import jax
import jax.numpy as jnp
from jax import lax
from jax.experimental import pallas as pl
from jax.experimental.pallas import tpu as pltpu
from jax.experimental.pallas import tpu_sc as plsc

NC = 2
NS = 16
LN = 16
CH = 128
ROWB = 1000


def _proj_body(x_ref, w_ref, b_ref, o_ref):
    o_ref[...] = (
        jnp.dot(x_ref[...], w_ref[...], preferred_element_type=jnp.float32)
        + b_ref[...]
    )


def _combine_proj_body(p_ref, s_ref, w_ref, b_ref, o_ref):
    p = p_ref[...]
    numer = p[0, :, :16] + p[1, :, :16]
    denom = (p[0, :, 16] + p[1, :, 16])[:, None]
    agg = jnp.where(denom > 0.0, numer / denom, 0.0)
    h = jnp.maximum(agg + s_ref[...], 0.0)
    o_ref[...] = (
        jnp.dot(h, w_ref[...], preferred_element_type=jnp.float32) + b_ref[...]
    )


def _combine_body(p_ref, s_ref, o_ref):
    p = p_ref[...]
    numer = p[0, :, :16] + p[1, :, :16]
    denom = (p[0, :, 16] + p[1, :, 16])[:, None]
    agg = jnp.where(denom > 0.0, numer / denom, 0.0)
    o_ref[...] = agg + s_ref[...]


def _edge_body(K, NACC, scale,
               qd_hbm, kv_hbm, dstg_hbm, dsts_hbm, src_hbm, zero_hbm,
               out_hbm,
               dg_v, ds_v, sr_v, qd_b, kv_b, pay_b, eb, acc, sem_q, sem_k):
    c = lax.axis_index("c")
    s = lax.axis_index("s")
    wid = s * NC + c
    rows_per_tile = NACC // NS
    r0 = s * rows_per_tile

    pltpu.sync_copy(zero_hbm.at[pl.ds(r0, rows_per_tile)],
                    acc.at[pl.ds(r0, rows_per_tile)])
    pltpu.sync_copy(dstg_hbm.at[pl.ds(wid * K, K)], dg_v)
    pltpu.sync_copy(dsts_hbm.at[pl.ds(wid * K, K)], ds_v)
    pltpu.sync_copy(src_hbm.at[pl.ds(wid * K, K)], sr_v)

    zero16 = jnp.zeros((LN,), jnp.float32)
    for b in range(CH // LN):
        rows = b * LN + lax.iota(jnp.int32, LN)
        for col in range(17, 32):
            cols = jnp.full((LN,), col, jnp.int32)
            plsc.store_scatter(pay_b, [rows, cols], zero16)

    plsc.subcore_barrier()

    def chunk_body(t, _):
        dg_row = dg_v.at[t]
        ds_row = ds_v.at[t]
        sr_row = sr_v.at[t]
        cp_q = pltpu.async_copy(qd_hbm.at[dg_row], qd_b, sem_q)
        cp_k = pltpu.async_copy(kv_hbm.at[sr_row], kv_b, sem_k)
        cp_q.wait()
        cp_k.wait()
        for b in range(CH // LN):
            rows = b * LN + lax.iota(jnp.int32, LN)
            dot = jnp.zeros((LN,), jnp.float32)
            for col in range(16):
                cols = jnp.full((LN,), col, jnp.int32)
                qc = plsc.load_gather(qd_b, [rows, cols])
                kc = plsc.load_gather(kv_b, [rows, cols])
                dot = dot + qc * kc
            e16 = jnp.exp(dot * scale)
            eb[pl.ds(b * LN, LN)] = e16
            for col in range(16):
                vcols = jnp.full((LN,), 16 + col, jnp.int32)
                vc = plsc.load_gather(kv_b, [rows, vcols])
                pcols = jnp.full((LN,), col, jnp.int32)
                plsc.store_scatter(pay_b, [rows, pcols], vc * e16)
            ecols = jnp.full((LN,), 16, jnp.int32)
            plsc.store_scatter(pay_b, [rows, ecols], e16)
        pltpu.sync_copy(pay_b, acc.at[ds_row], add=True)
        return _

    lax.fori_loop(0, K, chunk_body, None)

    plsc.subcore_barrier()
    pltpu.sync_copy(acc.at[pl.ds(r0, rows_per_tile)],
                    out_hbm.at[c, pl.ds(r0, rows_per_tile)])


def kernel(x, edge_index, Wq1, bq1, Wk1, bk1, Wv1, bv1, Ws1, bs1,
           Wq2, bq2, Wk2, bk2, Wv2, bv2, Ws2, bs2):
    N, D = x.shape
    H = Wq1.shape[1]
    C = Wq2.shape[1]
    E = edge_index.shape[1]
    NW = NC * NS
    K = -(-E // (NW * CH))
    K = ((K + 7) // 8) * 8
    Ep = NW * K * CH
    R = Ep // CH
    NACC = ((N + 1 + NS * 8 - 1) // (NS * 8)) * (NS * 8)
    scale = 1.0 / (float(H) ** 0.5)

    src = edge_index[0]
    dst = edge_index[1]
    pad = Ep - E
    srcp = jnp.concatenate([src, jnp.zeros((pad,), jnp.int32)])
    dstg = jnp.concatenate([dst, jnp.zeros((pad,), jnp.int32)])
    dsts = jnp.concatenate([dst, jnp.full((pad,), N, jnp.int32)])
    src2 = srcp.reshape(R, CH)
    dstg2 = dstg.reshape(R, CH)
    dsts2 = dsts.reshape(R, CH)
    zeros_acc = jnp.zeros((NACC, 32), jnp.float32)

    Wall1 = jnp.concatenate([Wq1, Wk1, Wv1, Ws1], axis=1)
    ball1 = jnp.concatenate([bq1, bk1, bv1, bs1])[None, :]
    Wall2 = jnp.concatenate([Wq2, Wk2, Wv2, Ws2], axis=1)
    ball2 = jnp.concatenate([bq2, bk2, bv2, bs2])[None, :]

    nblk = N // ROWB
    proj1 = pl.pallas_call(
        _proj_body,
        grid=(nblk,),
        in_specs=[
            pl.BlockSpec((ROWB, D), lambda i: (i, 0)),
            pl.BlockSpec((D, 4 * H), lambda i: (0, 0)),
            pl.BlockSpec((1, 4 * H), lambda i: (0, 0)),
        ],
        out_specs=pl.BlockSpec((ROWB, 4 * H), lambda i: (i, 0)),
        out_shape=jax.ShapeDtypeStruct((N, 4 * H), jnp.float32),
    )(x, Wall1, ball1)

    mesh = plsc.VectorSubcoreMesh(
        core_axis_name="c", subcore_axis_name="s",
        num_cores=NC, num_subcores=NS)
    edge_call = pl.kernel(
        functools.partial(_edge_body, K, NACC, scale),
        out_type=jax.ShapeDtypeStruct((NC, NACC, 32), jnp.float32),
        mesh=mesh,
        compiler_params=pltpu.CompilerParams(
            needs_layout_passes=False, use_tc_tiling_on_sc=False),
        scratch_types=[
            pltpu.VMEM((K, CH), jnp.int32),
            pltpu.VMEM((K, CH), jnp.int32),
            pltpu.VMEM((K, CH), jnp.int32),
            pltpu.VMEM((CH, 16), jnp.float32),
            pltpu.VMEM((CH, 32), jnp.float32),
            pltpu.VMEM((CH, 32), jnp.float32),
            pltpu.VMEM((CH,), jnp.float32),
            pltpu.VMEM_SHARED((NACC, 32), jnp.float32),
            pltpu.SemaphoreType.DMA,
            pltpu.SemaphoreType.DMA,
        ],
    )

    qd1 = proj1[:, :H]
    kv1 = proj1[:, H:3 * H]
    s1 = proj1[:, 3 * H:]
    parts1 = edge_call(qd1, kv1, dstg2, dsts2, src2, zeros_acc)

    proj2 = pl.pallas_call(
        _combine_proj_body,
        grid=(nblk,),
        in_specs=[
            pl.BlockSpec((NC, ROWB, 32), lambda i: (0, i, 0)),
            pl.BlockSpec((ROWB, H), lambda i: (i, 0)),
            pl.BlockSpec((H, 4 * C), lambda i: (0, 0)),
            pl.BlockSpec((1, 4 * C), lambda i: (0, 0)),
        ],
        out_specs=pl.BlockSpec((ROWB, 4 * C), lambda i: (i, 0)),
        out_shape=jax.ShapeDtypeStruct((N, 4 * C), jnp.float32),
    )(parts1[:, :N, :], s1, Wall2, ball2)

    qd2 = proj2[:, :C]
    kv2 = proj2[:, C:3 * C]
    s2 = proj2[:, 3 * C:]
    parts2 = edge_call(qd2, kv2, dstg2, dsts2, src2, zeros_acc)

    out = pl.pallas_call(
        _combine_body,
        grid=(nblk,),
        in_specs=[
            pl.BlockSpec((NC, ROWB, 32), lambda i: (0, i, 0)),
            pl.BlockSpec((ROWB, C), lambda i: (i, 0)),
        ],
        out_specs=pl.BlockSpec((ROWB, C), lambda i: (i, 0)),
        out_shape=jax.ShapeDtypeStruct((N, C), jnp.float32),
    )(parts2[:, :N, :], s2)
    return out


# speedup vs baseline: 17.9484x; 1.3817x over previous
import functools

import jax
import jax.numpy as jnp
from jax import lax
from jax.experimental import pallas as pl
from jax.experimental.pallas import tpu as pltpu
from jax.experimental.pallas import tpu_sc as plsc

NC = 2
NS = 16
LN = 16
CH = 128
ROWB = 1000


def _proj_body(x_ref, w_ref, b_ref, o_ref):
    o_ref[...] = (
        jnp.dot(x_ref[...], w_ref[...], preferred_element_type=jnp.float32)
        + b_ref[...]
    )


def _combine_proj_body(p_ref, s_ref, w_ref, b_ref, o_ref):
    p = p_ref[...]
    numer = p[0, :, :16] + p[1, :, :16]
    denom = (p[0, :, 16] + p[1, :, 16])[:, None]
    agg = jnp.where(denom > 0.0, numer / denom, 0.0)
    h = jnp.maximum(agg + s_ref[...], 0.0)
    o_ref[...] = (
        jnp.dot(h, w_ref[...], preferred_element_type=jnp.float32) + b_ref[...]
    )


def _combine_body(p_ref, s_ref, o_ref):
    p = p_ref[...]
    numer = p[0, :, :16] + p[1, :, :16]
    denom = (p[0, :, 16] + p[1, :, 16])[:, None]
    agg = jnp.where(denom > 0.0, numer / denom, 0.0)
    o_ref[...] = agg + s_ref[...]


NBUF = 4


def _edge_body(K, NACC, scale,
               qd_hbm, kv_hbm, dstg_hbm, dsts_hbm, src_hbm, zero_hbm,
               out_hbm,
               dg_v, ds_v, sr_v,
               qd0, qd1, qd2, qd3, kv0, kv1, kv2, kv3,
               pay0, pay1, pay2, pay3,
               acc, sem_gq, sem_gk, sem_s):
    qd_b = [qd0, qd1, qd2, qd3]
    kv_b = [kv0, kv1, kv2, kv3]
    pay_b = [pay0, pay1, pay2, pay3]
    c = lax.axis_index("c")
    s = lax.axis_index("s")
    wid = s * NC + c
    rows_per_tile = NACC // NS
    r0 = s * rows_per_tile

    pltpu.sync_copy(zero_hbm.at[pl.ds(r0, rows_per_tile)],
                    acc.at[pl.ds(r0, rows_per_tile)])
    pltpu.sync_copy(dstg_hbm.at[pl.ds(wid * K, K)], dg_v)
    pltpu.sync_copy(dsts_hbm.at[pl.ds(wid * K, K)], ds_v)
    pltpu.sync_copy(src_hbm.at[pl.ds(wid * K, K)], sr_v)

    zero16 = jnp.zeros((LN,), jnp.float32)
    for p in range(NBUF):
        for b in range(CH // LN):
            rows = b * LN + lax.iota(jnp.int32, LN)
            for col in range(17, 32):
                cols = jnp.full((LN,), col, jnp.int32)
                plsc.store_scatter(pay_b[p], [rows, cols], zero16)

    plsc.subcore_barrier()

    for p in range(NBUF):
        pltpu.async_copy(qd_hbm.at[dg_v.at[p]], qd_b[p], sem_gq.at[p])
        pltpu.async_copy(kv_hbm.at[sr_v.at[p]], kv_b[p], sem_gk.at[p])

    T4 = K // NBUF

    def outer(t4, _):
        for p in range(NBUF):
            t = t4 * NBUF + p
            pltpu.make_async_copy(
                qd_hbm.at[dg_v.at[t]], qd_b[p], sem_gq.at[p]).wait()
            pltpu.make_async_copy(
                kv_hbm.at[sr_v.at[t]], kv_b[p], sem_gk.at[p]).wait()

            @pl.when(t4 > 0)
            def _wait_scatter():
                pltpu.make_async_copy(
                    pay_b[p], acc.at[ds_v.at[t]], sem_s.at[p]).wait()

            for b in range(CH // LN):
                rows = b * LN + lax.iota(jnp.int32, LN)
                dot = jnp.zeros((LN,), jnp.float32)
                for col in range(16):
                    cols = jnp.full((LN,), col, jnp.int32)
                    qc = plsc.load_gather(qd_b[p], [rows, cols])
                    kc = plsc.load_gather(kv_b[p], [rows, cols])
                    dot = dot + qc * kc
                e16 = jnp.exp(dot * scale)
                for col in range(16):
                    vcols = jnp.full((LN,), 16 + col, jnp.int32)
                    vc = plsc.load_gather(kv_b[p], [rows, vcols])
                    pcols = jnp.full((LN,), col, jnp.int32)
                    plsc.store_scatter(pay_b[p], [rows, pcols], vc * e16)
                ecols = jnp.full((LN,), 16, jnp.int32)
                plsc.store_scatter(pay_b[p], [rows, ecols], e16)

            pltpu.async_copy(
                pay_b[p], acc.at[ds_v.at[t]], sem_s.at[p], add=True)

            @pl.when(t4 < T4 - 1)
            def _prefetch():
                tn = t + NBUF
                pltpu.async_copy(
                    qd_hbm.at[dg_v.at[tn]], qd_b[p], sem_gq.at[p])
                pltpu.async_copy(
                    kv_hbm.at[sr_v.at[tn]], kv_b[p], sem_gk.at[p])
        return _

    lax.fori_loop(0, T4, outer, None)

    for p in range(NBUF):
        t = K - NBUF + p
        pltpu.make_async_copy(
            pay_b[p], acc.at[ds_v.at[t]], sem_s.at[p]).wait()

    plsc.subcore_barrier()
    pltpu.sync_copy(acc.at[pl.ds(r0, rows_per_tile)],
                    out_hbm.at[c, pl.ds(r0, rows_per_tile)])


def kernel(x, edge_index, Wq1, bq1, Wk1, bk1, Wv1, bv1, Ws1, bs1,
           Wq2, bq2, Wk2, bk2, Wv2, bv2, Ws2, bs2):
    N, D = x.shape
    H = Wq1.shape[1]
    C = Wq2.shape[1]
    E = edge_index.shape[1]
    NW = NC * NS
    K = -(-E // (NW * CH))
    K = ((K + 7) // 8) * 8
    Ep = NW * K * CH
    R = Ep // CH
    NACC = ((N + 1 + NS * 8 - 1) // (NS * 8)) * (NS * 8)
    scale = 1.0 / (float(H) ** 0.5)

    src = edge_index[0]
    dst = edge_index[1]
    pad = Ep - E
    srcp = jnp.concatenate([src, jnp.zeros((pad,), jnp.int32)])
    dstg = jnp.concatenate([dst, jnp.zeros((pad,), jnp.int32)])
    dsts = jnp.concatenate([dst, jnp.full((pad,), N, jnp.int32)])
    src2 = srcp.reshape(R, CH)
    dstg2 = dstg.reshape(R, CH)
    dsts2 = dsts.reshape(R, CH)
    zeros_acc = jnp.zeros((NACC, 32), jnp.float32)

    Wall1 = jnp.concatenate([Wq1, Wk1, Wv1, Ws1], axis=1)
    ball1 = jnp.concatenate([bq1, bk1, bv1, bs1])[None, :]
    Wall2 = jnp.concatenate([Wq2, Wk2, Wv2, Ws2], axis=1)
    ball2 = jnp.concatenate([bq2, bk2, bv2, bs2])[None, :]

    nblk = N // ROWB
    proj1 = pl.pallas_call(
        _proj_body,
        grid=(nblk,),
        in_specs=[
            pl.BlockSpec((ROWB, D), lambda i: (i, 0)),
            pl.BlockSpec((D, 4 * H), lambda i: (0, 0)),
            pl.BlockSpec((1, 4 * H), lambda i: (0, 0)),
        ],
        out_specs=pl.BlockSpec((ROWB, 4 * H), lambda i: (i, 0)),
        out_shape=jax.ShapeDtypeStruct((N, 4 * H), jnp.float32),
    )(x, Wall1, ball1)

    mesh = plsc.VectorSubcoreMesh(
        core_axis_name="c", subcore_axis_name="s",
        num_cores=NC, num_subcores=NS)
    edge_call = pl.kernel(
        functools.partial(_edge_body, K, NACC, scale),
        out_type=jax.ShapeDtypeStruct((NC, NACC, 32), jnp.float32),
        mesh=mesh,
        compiler_params=pltpu.CompilerParams(
            needs_layout_passes=False, use_tc_tiling_on_sc=False),
        scratch_types=(
            [
                pltpu.VMEM((K, CH), jnp.int32),
                pltpu.VMEM((K, CH), jnp.int32),
                pltpu.VMEM((K, CH), jnp.int32),
            ]
            + [pltpu.VMEM((CH, 16), jnp.float32) for _ in range(NBUF)]
            + [pltpu.VMEM((CH, 32), jnp.float32) for _ in range(NBUF)]
            + [pltpu.VMEM((CH, 32), jnp.float32) for _ in range(NBUF)]
            + [
                pltpu.VMEM_SHARED((NACC, 32), jnp.float32),
                pltpu.SemaphoreType.DMA((NBUF,)),
                pltpu.SemaphoreType.DMA((NBUF,)),
                pltpu.SemaphoreType.DMA((NBUF,)),
            ]
        ),
    )

    qd1 = proj1[:, :H]
    kv1 = proj1[:, H:3 * H]
    s1 = proj1[:, 3 * H:]
    parts1 = edge_call(qd1, kv1, dstg2, dsts2, src2, zeros_acc)

    proj2 = pl.pallas_call(
        _combine_proj_body,
        grid=(nblk,),
        in_specs=[
            pl.BlockSpec((NC, ROWB, 32), lambda i: (0, i, 0)),
            pl.BlockSpec((ROWB, H), lambda i: (i, 0)),
            pl.BlockSpec((H, 4 * C), lambda i: (0, 0)),
            pl.BlockSpec((1, 4 * C), lambda i: (0, 0)),
        ],
        out_specs=pl.BlockSpec((ROWB, 4 * C), lambda i: (i, 0)),
        out_shape=jax.ShapeDtypeStruct((N, 4 * C), jnp.float32),
    )(parts1[:, :N, :], s1, Wall2, ball2)

    qd2 = proj2[:, :C]
    kv2 = proj2[:, C:3 * C]
    s2 = proj2[:, 3 * C:]
    parts2 = edge_call(qd2, kv2, dstg2, dsts2, src2, zeros_acc)

    out = pl.pallas_call(
        _combine_body,
        grid=(nblk,),
        in_specs=[
            pl.BlockSpec((NC, ROWB, 32), lambda i: (0, i, 0)),
            pl.BlockSpec((ROWB, C), lambda i: (i, 0)),
        ],
        out_specs=pl.BlockSpec((ROWB, C), lambda i: (i, 0)),
        out_shape=jax.ShapeDtypeStruct((N, C), jnp.float32),
    )(parts2[:, :N, :], s2)
    return out


# speedup vs baseline: 20.1209x; 1.1210x over previous
import functools

import jax
import jax.numpy as jnp
from jax import lax
from jax.experimental import pallas as pl
from jax.experimental.pallas import tpu as pltpu
from jax.experimental.pallas import tpu_sc as plsc

NC = 2
NS = 16
LN = 16
CH = 128
ROWB = 1000


def _proj_body(x_ref, w_ref, b_ref, o_ref):
    o_ref[...] = (
        jnp.dot(x_ref[...], w_ref[...], preferred_element_type=jnp.float32)
        + b_ref[...]
    )


def _combine_proj_body(p_ref, d_ref, s_ref, w_ref, b_ref, o_ref):
    p = p_ref[...]
    numer = p[0] + p[1]
    denom = jnp.sum(d_ref[...], axis=1)[:, None]
    agg = jnp.where(denom > 0.0, numer / denom, 0.0)
    h = jnp.maximum(agg + s_ref[...], 0.0)
    o_ref[...] = (
        jnp.dot(h, w_ref[...], preferred_element_type=jnp.float32) + b_ref[...]
    )


def _combine_body(p_ref, d_ref, s_ref, o_ref):
    p = p_ref[...]
    numer = p[0] + p[1]
    denom = jnp.sum(d_ref[...], axis=1)[:, None]
    agg = jnp.where(denom > 0.0, numer / denom, 0.0)
    o_ref[...] = agg + s_ref[...]


NBUF = 4


def _edge_body(K, NACC, scale,
               qd_hbm, kv_hbm, dstg_hbm, dsts_hbm, src_hbm, zero_hbm,
               out_hbm, outd_hbm,
               dg_v, ds_v, sr_v,
               qd0, qd1, qd2, qd3, kv0, kv1, kv2, kv3,
               pay0, pay1, pay2, pay3,
               dvec, acc, sem_gq, sem_gk, sem_s):
    qd_b = [qd0, qd1, qd2, qd3]
    kv_b = [kv0, kv1, kv2, kv3]
    pay_b = [pay0, pay1, pay2, pay3]
    c = lax.axis_index("c")
    s = lax.axis_index("s")
    wid = s * NC + c
    rows_per_tile = NACC // NS
    r0 = s * rows_per_tile

    pltpu.sync_copy(zero_hbm.at[pl.ds(r0, rows_per_tile)],
                    acc.at[pl.ds(r0, rows_per_tile)])
    zero16 = jnp.zeros((LN,), jnp.float32)

    def zero_body(i, _):
        dvec[pl.ds(i * LN, LN)] = zero16
        return _

    lax.fori_loop(0, NACC // LN, zero_body, None)
    pltpu.sync_copy(dstg_hbm.at[pl.ds(wid * K, K)], dg_v)
    pltpu.sync_copy(dsts_hbm.at[pl.ds(wid * K, K)], ds_v)
    pltpu.sync_copy(src_hbm.at[pl.ds(wid * K, K)], sr_v)

    plsc.subcore_barrier()

    for p in range(NBUF):
        pltpu.async_copy(qd_hbm.at[dg_v.at[p]], qd_b[p], sem_gq.at[p])
        pltpu.async_copy(kv_hbm.at[sr_v.at[p]], kv_b[p], sem_gk.at[p])

    T4 = K // NBUF

    def outer(t4, _):
        for p in range(NBUF):
            t = t4 * NBUF + p
            pltpu.make_async_copy(
                qd_hbm.at[dg_v.at[t]], qd_b[p], sem_gq.at[p]).wait()
            pltpu.make_async_copy(
                kv_hbm.at[sr_v.at[t]], kv_b[p], sem_gk.at[p]).wait()

            @pl.when(t4 > 0)
            def _wait_scatter():
                pltpu.make_async_copy(
                    pay_b[p], acc.at[ds_v.at[t]], sem_s.at[p]).wait()

            for b in range(CH // LN):
                rows = b * LN + lax.iota(jnp.int32, LN)
                dot = jnp.zeros((LN,), jnp.float32)
                for col in range(16):
                    cols = jnp.full((LN,), col, jnp.int32)
                    qc = plsc.load_gather(qd_b[p], [rows, cols])
                    kc = plsc.load_gather(kv_b[p], [rows, cols])
                    dot = dot + qc * kc
                e16 = jnp.exp(dot * scale)
                for col in range(16):
                    vcols = jnp.full((LN,), 16 + col, jnp.int32)
                    vc = plsc.load_gather(kv_b[p], [rows, vcols])
                    pcols = jnp.full((LN,), col, jnp.int32)
                    plsc.store_scatter(pay_b[p], [rows, pcols], vc * e16)
                dst16 = ds_v[t, pl.ds(b * LN, LN)]
                plsc.addupdate_scatter(dvec, [dst16], e16)

            pltpu.async_copy(
                pay_b[p], acc.at[ds_v.at[t]], sem_s.at[p], add=True)

            @pl.when(t4 < T4 - 1)
            def _prefetch():
                tn = t + NBUF
                pltpu.async_copy(
                    qd_hbm.at[dg_v.at[tn]], qd_b[p], sem_gq.at[p])
                pltpu.async_copy(
                    kv_hbm.at[sr_v.at[tn]], kv_b[p], sem_gk.at[p])
        return _

    lax.fori_loop(0, T4, outer, None)

    for p in range(NBUF):
        t = K - NBUF + p
        pltpu.make_async_copy(
            pay_b[p], acc.at[ds_v.at[t]], sem_s.at[p]).wait()

    plsc.subcore_barrier()
    pltpu.sync_copy(acc.at[pl.ds(r0, rows_per_tile)],
                    out_hbm.at[c, pl.ds(r0, rows_per_tile)])
    pltpu.sync_copy(dvec, outd_hbm.at[c, s])


def kernel(x, edge_index, Wq1, bq1, Wk1, bk1, Wv1, bv1, Ws1, bs1,
           Wq2, bq2, Wk2, bk2, Wv2, bv2, Ws2, bs2):
    N, D = x.shape
    H = Wq1.shape[1]
    C = Wq2.shape[1]
    E = edge_index.shape[1]
    NW = NC * NS
    K = -(-E // (NW * CH))
    K = ((K + 7) // 8) * 8
    Ep = NW * K * CH
    R = Ep // CH
    NACC = ((N + 1 + NS * 8 - 1) // (NS * 8)) * (NS * 8)
    scale = 1.0 / (float(H) ** 0.5)

    src = edge_index[0]
    dst = edge_index[1]
    pad = Ep - E
    srcp = jnp.concatenate([src, jnp.zeros((pad,), jnp.int32)])
    dstg = jnp.concatenate([dst, jnp.zeros((pad,), jnp.int32)])
    pad_rows = N + jnp.arange(pad, dtype=jnp.int32) % (NACC - N)
    dsts = jnp.concatenate([dst, pad_rows])
    src2 = srcp.reshape(R, CH)
    dstg2 = dstg.reshape(R, CH)
    dsts2 = dsts.reshape(R, CH)
    zeros_acc = jnp.zeros((NACC, 16), jnp.float32)

    Wall1 = jnp.concatenate([Wq1, Wk1, Wv1, Ws1], axis=1)
    ball1 = jnp.concatenate([bq1, bk1, bv1, bs1])[None, :]
    Wall2 = jnp.concatenate([Wq2, Wk2, Wv2, Ws2], axis=1)
    ball2 = jnp.concatenate([bq2, bk2, bv2, bs2])[None, :]

    nblk = N // ROWB
    proj1 = pl.pallas_call(
        _proj_body,
        grid=(nblk,),
        in_specs=[
            pl.BlockSpec((ROWB, D), lambda i: (i, 0)),
            pl.BlockSpec((D, 4 * H), lambda i: (0, 0)),
            pl.BlockSpec((1, 4 * H), lambda i: (0, 0)),
        ],
        out_specs=pl.BlockSpec((ROWB, 4 * H), lambda i: (i, 0)),
        out_shape=jax.ShapeDtypeStruct((N, 4 * H), jnp.float32),
    )(x, Wall1, ball1)

    mesh = plsc.VectorSubcoreMesh(
        core_axis_name="c", subcore_axis_name="s",
        num_cores=NC, num_subcores=NS)
    edge_call = pl.kernel(
        functools.partial(_edge_body, K, NACC, scale),
        out_type=(
            jax.ShapeDtypeStruct((NC, NACC, 16), jnp.float32),
            jax.ShapeDtypeStruct((NC, NS, NACC), jnp.float32),
        ),
        mesh=mesh,
        compiler_params=pltpu.CompilerParams(
            needs_layout_passes=False, use_tc_tiling_on_sc=False),
        scratch_types=(
            [
                pltpu.VMEM((K, CH), jnp.int32),
                pltpu.VMEM((K, CH), jnp.int32),
                pltpu.VMEM((K, CH), jnp.int32),
            ]
            + [pltpu.VMEM((CH, 16), jnp.float32) for _ in range(NBUF)]
            + [pltpu.VMEM((CH, 32), jnp.float32) for _ in range(NBUF)]
            + [pltpu.VMEM((CH, 16), jnp.float32) for _ in range(NBUF)]
            + [
                pltpu.VMEM((NACC,), jnp.float32),
                pltpu.VMEM_SHARED((NACC, 16), jnp.float32),
                pltpu.SemaphoreType.DMA((NBUF,)),
                pltpu.SemaphoreType.DMA((NBUF,)),
                pltpu.SemaphoreType.DMA((NBUF,)),
            ]
        ),
    )

    qd1 = proj1[:, :H]
    kv1 = proj1[:, H:3 * H]
    s1 = proj1[:, 3 * H:]
    parts1, dparts1 = edge_call(qd1, kv1, dstg2, dsts2, src2, zeros_acc)

    proj2 = pl.pallas_call(
        _combine_proj_body,
        grid=(nblk,),
        in_specs=[
            pl.BlockSpec((NC, ROWB, H), lambda i: (0, i, 0)),
            pl.BlockSpec((ROWB, NC * NS), lambda i: (i, 0)),
            pl.BlockSpec((ROWB, H), lambda i: (i, 0)),
            pl.BlockSpec((H, 4 * C), lambda i: (0, 0)),
            pl.BlockSpec((1, 4 * C), lambda i: (0, 0)),
        ],
        out_specs=pl.BlockSpec((ROWB, 4 * C), lambda i: (i, 0)),
        out_shape=jax.ShapeDtypeStruct((N, 4 * C), jnp.float32),
    )(parts1[:, :N, :], dparts1.reshape(NC * NS, NACC).T[:N], s1, Wall2, ball2)

    qd2 = proj2[:, :C]
    kv2 = proj2[:, C:3 * C]
    s2 = proj2[:, 3 * C:]
    parts2, dparts2 = edge_call(qd2, kv2, dstg2, dsts2, src2, zeros_acc)

    out = pl.pallas_call(
        _combine_body,
        grid=(nblk,),
        in_specs=[
            pl.BlockSpec((NC, ROWB, C), lambda i: (0, i, 0)),
            pl.BlockSpec((ROWB, NC * NS), lambda i: (i, 0)),
            pl.BlockSpec((ROWB, C), lambda i: (i, 0)),
        ],
        out_specs=pl.BlockSpec((ROWB, C), lambda i: (i, 0)),
        out_shape=jax.ShapeDtypeStruct((N, C), jnp.float32),
    )(parts2[:, :N, :], dparts2.reshape(NC * NS, NACC).T[:N], s2)
    return out


# speedup vs baseline: 20.1752x; 1.0027x over previous
import functools

import jax
import jax.numpy as jnp
from jax import lax
from jax.experimental import pallas as pl
from jax.experimental.pallas import tpu as pltpu
from jax.experimental.pallas import tpu_sc as plsc

NC = 2
NS = 16
LN = 16
CH = 128
ROWB = 1000


def _proj_body(x_ref, w_ref, b_ref, o_ref):
    o_ref[...] = (
        jnp.dot(x_ref[...], w_ref[...], preferred_element_type=jnp.float32)
        + b_ref[...]
    )


def _combine_proj_body(p_ref, d_ref, s_ref, w_ref, b_ref, o_ref):
    p = p_ref[...]
    numer = p[0] + p[1]
    denom = jnp.sum(d_ref[...], axis=1)[:, None]
    agg = jnp.where(denom > 0.0, numer / denom, 0.0)
    h = jnp.maximum(agg + s_ref[...], 0.0)
    o_ref[...] = (
        jnp.dot(h, w_ref[...], preferred_element_type=jnp.float32) + b_ref[...]
    )


def _combine_body(p_ref, d_ref, s_ref, o_ref):
    p = p_ref[...]
    numer = p[0] + p[1]
    denom = jnp.sum(d_ref[...], axis=1)[:, None]
    agg = jnp.where(denom > 0.0, numer / denom, 0.0)
    o_ref[...] = agg + s_ref[...]


NBUF = 8


def _edge_body(K, NACC, scale,
               qd_hbm, kv_hbm, dstg_hbm, dsts_hbm, src_hbm, zero_hbm,
               out_hbm, outd_hbm,
               dg_v, ds_v, sr_v,
               qd0, qd1, qd2, qd3, qd4, qd5, qd6, qd7,
               kv0, kv1, kv2, kv3, kv4, kv5, kv6, kv7,
               pay0, pay1, pay2, pay3, pay4, pay5, pay6, pay7,
               dvec, acc, sem_gq, sem_gk, sem_s):
    qd_b = [qd0, qd1, qd2, qd3, qd4, qd5, qd6, qd7]
    kv_b = [kv0, kv1, kv2, kv3, kv4, kv5, kv6, kv7]
    pay_b = [pay0, pay1, pay2, pay3, pay4, pay5, pay6, pay7]
    c = lax.axis_index("c")
    s = lax.axis_index("s")
    wid = s * NC + c
    rows_per_tile = NACC // NS
    r0 = s * rows_per_tile

    pltpu.sync_copy(zero_hbm.at[pl.ds(r0, rows_per_tile)],
                    acc.at[pl.ds(r0, rows_per_tile)])
    zero16 = jnp.zeros((LN,), jnp.float32)

    def zero_body(i, _):
        dvec[pl.ds(i * LN, LN)] = zero16
        return _

    lax.fori_loop(0, NACC // LN, zero_body, None)
    pltpu.sync_copy(dstg_hbm.at[pl.ds(wid * K, K)], dg_v)
    pltpu.sync_copy(dsts_hbm.at[pl.ds(wid * K, K)], ds_v)
    pltpu.sync_copy(src_hbm.at[pl.ds(wid * K, K)], sr_v)

    plsc.subcore_barrier()

    for p in range(NBUF):
        pltpu.async_copy(qd_hbm.at[dg_v.at[p]], qd_b[p], sem_gq.at[p])
        pltpu.async_copy(kv_hbm.at[sr_v.at[p]], kv_b[p], sem_gk.at[p])

    T4 = K // NBUF

    def outer(t4, _):
        for p in range(NBUF):
            t = t4 * NBUF + p
            pltpu.make_async_copy(
                qd_hbm.at[dg_v.at[t]], qd_b[p], sem_gq.at[p]).wait()
            pltpu.make_async_copy(
                kv_hbm.at[sr_v.at[t]], kv_b[p], sem_gk.at[p]).wait()

            @pl.when(t4 > 0)
            def _wait_scatter():
                pltpu.make_async_copy(
                    pay_b[p], acc.at[ds_v.at[t]], sem_s.at[p]).wait()

            for b in range(CH // LN):
                rows = b * LN + lax.iota(jnp.int32, LN)
                dot = jnp.zeros((LN,), jnp.float32)
                for col in range(16):
                    cols = jnp.full((LN,), col, jnp.int32)
                    qc = plsc.load_gather(qd_b[p], [rows, cols])
                    kc = plsc.load_gather(kv_b[p], [rows, cols])
                    dot = dot + qc * kc
                e16 = jnp.exp(dot * scale)
                for col in range(16):
                    vcols = jnp.full((LN,), 16 + col, jnp.int32)
                    vc = plsc.load_gather(kv_b[p], [rows, vcols])
                    pcols = jnp.full((LN,), col, jnp.int32)
                    plsc.store_scatter(pay_b[p], [rows, pcols], vc * e16)
                dst16 = ds_v[t, pl.ds(b * LN, LN)]
                plsc.addupdate_scatter(dvec, [dst16], e16)

            pltpu.async_copy(
                pay_b[p], acc.at[ds_v.at[t]], sem_s.at[p], add=True)

            @pl.when(t4 < T4 - 1)
            def _prefetch():
                tn = t + NBUF
                pltpu.async_copy(
                    qd_hbm.at[dg_v.at[tn]], qd_b[p], sem_gq.at[p])
                pltpu.async_copy(
                    kv_hbm.at[sr_v.at[tn]], kv_b[p], sem_gk.at[p])
        return _

    lax.fori_loop(0, T4, outer, None)

    for p in range(NBUF):
        t = K - NBUF + p
        pltpu.make_async_copy(
            pay_b[p], acc.at[ds_v.at[t]], sem_s.at[p]).wait()

    plsc.subcore_barrier()
    pltpu.sync_copy(acc.at[pl.ds(r0, rows_per_tile)],
                    out_hbm.at[c, pl.ds(r0, rows_per_tile)])
    pltpu.sync_copy(dvec, outd_hbm.at[c, s])


def kernel(x, edge_index, Wq1, bq1, Wk1, bk1, Wv1, bv1, Ws1, bs1,
           Wq2, bq2, Wk2, bk2, Wv2, bv2, Ws2, bs2):
    N, D = x.shape
    H = Wq1.shape[1]
    C = Wq2.shape[1]
    E = edge_index.shape[1]
    NW = NC * NS
    K = -(-E // (NW * CH))
    K = ((K + 7) // 8) * 8
    Ep = NW * K * CH
    R = Ep // CH
    NACC = ((N + 1 + NS * 8 - 1) // (NS * 8)) * (NS * 8)
    scale = 1.0 / (float(H) ** 0.5)

    src = edge_index[0]
    dst = edge_index[1]
    pad = Ep - E
    srcp = jnp.concatenate([src, jnp.zeros((pad,), jnp.int32)])
    dstg = jnp.concatenate([dst, jnp.zeros((pad,), jnp.int32)])
    pad_rows = N + jnp.arange(pad, dtype=jnp.int32) % (NACC - N)
    dsts = jnp.concatenate([dst, pad_rows])
    src2 = srcp.reshape(R, CH)
    dstg2 = dstg.reshape(R, CH)
    dsts2 = dsts.reshape(R, CH)
    zeros_acc = jnp.zeros((NACC, 16), jnp.float32)

    Wall1 = jnp.concatenate([Wq1, Wk1, Wv1, Ws1], axis=1)
    ball1 = jnp.concatenate([bq1, bk1, bv1, bs1])[None, :]
    Wall2 = jnp.concatenate([Wq2, Wk2, Wv2, Ws2], axis=1)
    ball2 = jnp.concatenate([bq2, bk2, bv2, bs2])[None, :]

    nblk = N // ROWB
    proj1 = pl.pallas_call(
        _proj_body,
        grid=(nblk,),
        in_specs=[
            pl.BlockSpec((ROWB, D), lambda i: (i, 0)),
            pl.BlockSpec((D, 4 * H), lambda i: (0, 0)),
            pl.BlockSpec((1, 4 * H), lambda i: (0, 0)),
        ],
        out_specs=pl.BlockSpec((ROWB, 4 * H), lambda i: (i, 0)),
        out_shape=jax.ShapeDtypeStruct((N, 4 * H), jnp.float32),
    )(x, Wall1, ball1)

    mesh = plsc.VectorSubcoreMesh(
        core_axis_name="c", subcore_axis_name="s",
        num_cores=NC, num_subcores=NS)
    edge_call = pl.kernel(
        functools.partial(_edge_body, K, NACC, scale),
        out_type=(
            jax.ShapeDtypeStruct((NC, NACC, 16), jnp.float32),
            jax.ShapeDtypeStruct((NC, NS, NACC), jnp.float32),
        ),
        mesh=mesh,
        compiler_params=pltpu.CompilerParams(
            needs_layout_passes=False, use_tc_tiling_on_sc=False),
        scratch_types=(
            [
                pltpu.VMEM((K, CH), jnp.int32),
                pltpu.VMEM((K, CH), jnp.int32),
                pltpu.VMEM((K, CH), jnp.int32),
            ]
            + [pltpu.VMEM((CH, 16), jnp.float32) for _ in range(NBUF)]
            + [pltpu.VMEM((CH, 32), jnp.float32) for _ in range(NBUF)]
            + [pltpu.VMEM((CH, 16), jnp.float32) for _ in range(NBUF)]
            + [
                pltpu.VMEM((NACC,), jnp.float32),
                pltpu.VMEM_SHARED((NACC, 16), jnp.float32),
                pltpu.SemaphoreType.DMA((NBUF,)),
                pltpu.SemaphoreType.DMA((NBUF,)),
                pltpu.SemaphoreType.DMA((NBUF,)),
            ]
        ),
    )

    qd1 = proj1[:, :H]
    kv1 = proj1[:, H:3 * H]
    s1 = proj1[:, 3 * H:]
    parts1, dparts1 = edge_call(qd1, kv1, dstg2, dsts2, src2, zeros_acc)

    proj2 = pl.pallas_call(
        _combine_proj_body,
        grid=(nblk,),
        in_specs=[
            pl.BlockSpec((NC, ROWB, H), lambda i: (0, i, 0)),
            pl.BlockSpec((ROWB, NC * NS), lambda i: (i, 0)),
            pl.BlockSpec((ROWB, H), lambda i: (i, 0)),
            pl.BlockSpec((H, 4 * C), lambda i: (0, 0)),
            pl.BlockSpec((1, 4 * C), lambda i: (0, 0)),
        ],
        out_specs=pl.BlockSpec((ROWB, 4 * C), lambda i: (i, 0)),
        out_shape=jax.ShapeDtypeStruct((N, 4 * C), jnp.float32),
    )(parts1[:, :N, :], dparts1.reshape(NC * NS, NACC).T[:N], s1, Wall2, ball2)

    qd2 = proj2[:, :C]
    kv2 = proj2[:, C:3 * C]
    s2 = proj2[:, 3 * C:]
    parts2, dparts2 = edge_call(qd2, kv2, dstg2, dsts2, src2, zeros_acc)

    out = pl.pallas_call(
        _combine_body,
        grid=(nblk,),
        in_specs=[
            pl.BlockSpec((NC, ROWB, C), lambda i: (0, i, 0)),
            pl.BlockSpec((ROWB, NC * NS), lambda i: (i, 0)),
            pl.BlockSpec((ROWB, C), lambda i: (i, 0)),
        ],
        out_specs=pl.BlockSpec((ROWB, C), lambda i: (i, 0)),
        out_shape=jax.ShapeDtypeStruct((N, C), jnp.float32),
    )(parts2[:, :N, :], dparts2.reshape(NC * NS, NACC).T[:N], s2)
    return out
